# baseline (device time: 45317 ns/iter reference)
import jax
import jax.numpy as jnp
from jax import lax
from jax.experimental import pallas as pl
from jax.experimental.pallas import tpu as pltpu

N_DEV = 8
M_BLK = 512
K_BLK = 512

ORDER = [4, 1, 7, 3, 2, 6, 5]


def kernel(x, w_mat):
    m_glob, k_per = x.shape
    k_glob, n = w_mat.shape

    def body(x_ref, w_ref, out_ref,
             xb_ref, xq_send_ref, xq_ref, sc_send_ref, sc_ref,
             wv_ref, wb_ref, xs_ref, amax_ref,
             send_sems, recv_sems, sc_send_sems, sc_recv_sems,
             w_sems, xs_sems, ax_send_sems, ax_recv_sems):
        my = lax.axis_index("i")

        barrier = pltpu.get_barrier_semaphore()
        for k in range(1, N_DEV):
            pl.semaphore_signal(
                barrier, inc=1,
                device_id=(lax.rem(my + k, N_DEV),),
                device_id_type=pl.DeviceIdType.MESH,
            )

        dsts = [lax.rem(my + k, N_DEV) for k in ORDER]
        srcs = [my] + [lax.rem(my - k + N_DEV, N_DEV) for k in ORDER]

        def w_copy(j):
            return pltpu.make_async_copy(
                w_ref.at[pl.ds(srcs[j] * K_BLK, K_BLK), :],
                wv_ref.at[j % 2],
                w_sems.at[j % 2],
            )

        w_copy(0).start()
        w_copy(1).start()

        xblocks = dsts + [my]

        def x_copy(t):
            return pltpu.make_async_copy(
                x_ref.at[pl.ds(xblocks[t] * M_BLK, M_BLK), :],
                xs_ref.at[t % 4],
                xs_sems.at[t % 4],
            )

        for t0 in range(4):
            x_copy(t0).start()

        pl.semaphore_wait(barrier, N_DEV - 1)

        sends = []
        for t in range(N_DEV):
            with jax.named_scope(f"stage#t={t}"):
                x_copy(t).wait()
                if t < N_DEV - 1:
                    tile = xs_ref[t % 4]
                    amax_c = jnp.maximum(
                        jnp.max(jnp.abs(tile), axis=0, keepdims=True), 1e-30)
                    xq_send_ref[t] = jnp.clip(
                        jnp.round(tile * (127.0 / amax_c)), -127.0, 127.0
                    ).astype(jnp.int8)
                    sc_send_ref[t] = amax_c * (1.0 / 127.0)
                else:
                    xb_ref[...] = xs_ref[t % 4].astype(jnp.bfloat16)
                if t + 4 < N_DEV:
                    x_copy(t + 4).start()
            if t < N_DEV - 1:
                with jax.named_scope(f"issue#t={t}"):
                    d = pltpu.make_async_remote_copy(
                        src_ref=xq_send_ref.at[t],
                        dst_ref=xq_ref.at[my],
                        send_sem=send_sems.at[t],
                        recv_sem=recv_sems.at[t],
                        device_id=(dsts[t],),
                        device_id_type=pl.DeviceIdType.MESH,
                    )
                    d.start()
                    sends.append(d)
                    s = pltpu.make_async_remote_copy(
                        src_ref=sc_send_ref.at[t],
                        dst_ref=sc_ref.at[my],
                        send_sem=sc_send_sems.at[t],
                        recv_sem=sc_recv_sems.at[t],
                        device_id=(dsts[t],),
                        device_id_type=pl.DeviceIdType.MESH,
                    )
                    s.start()
                    sends.append(s)

        with jax.named_scope("wprep#j=0"):
            w_copy(0).wait()
            wb_ref[0] = wv_ref[0].astype(jnp.bfloat16)

        acc = jnp.zeros((M_BLK, n), jnp.float32)
        for j in range(N_DEV):
            if j == 0:
                x_tile = xb_ref[...]
            else:
                with jax.named_scope(f"recv#j={j}"):
                    recv = pltpu.make_async_remote_copy(
                        src_ref=xq_send_ref.at[j - 1],
                        dst_ref=xq_ref.at[srcs[j]],
                        send_sem=send_sems.at[j - 1],
                        recv_sem=recv_sems.at[j - 1],
                        device_id=(srcs[j],),
                        device_id_type=pl.DeviceIdType.MESH,
                    )
                    recv.wait_recv()
                    screcv = pltpu.make_async_remote_copy(
                        src_ref=sc_send_ref.at[j - 1],
                        dst_ref=sc_ref.at[srcs[j]],
                        send_sem=sc_send_sems.at[j - 1],
                        recv_sem=sc_recv_sems.at[j - 1],
                        device_id=(srcs[j],),
                        device_id_type=pl.DeviceIdType.MESH,
                    )
                    screcv.wait_recv()
                with jax.named_scope(f"deq#j={j}"):
                    scale_row = sc_ref[srcs[j]]
                    x_tile = (
                        xq_ref[srcs[j]].astype(jnp.float32) * scale_row
                    ).astype(jnp.bfloat16)
            with jax.named_scope(f"dot#j={j}"):
                acc = acc + jnp.dot(
                    x_tile,
                    wb_ref[j % 2],
                    preferred_element_type=jnp.float32,
                )
            if j + 1 < N_DEV:
                with jax.named_scope(f"wprep#j={j + 1}"):
                    w_copy(j + 1).wait()
                    wb_ref[(j + 1) % 2] = (
                        wv_ref[(j + 1) % 2].astype(jnp.bfloat16))
                    if j + 2 < N_DEV:
                        w_copy(j + 2).start()

        with jax.named_scope("amax_send"):
            amax_ref[pl.ds(my, 1), :] = jnp.full(
                (1, 128), jnp.maximum(jnp.max(acc), 0.0), jnp.float32)
            ax_sends = []
            for k in range(1, N_DEV):
                dst = lax.rem(my + k, N_DEV)
                d = pltpu.make_async_remote_copy(
                    src_ref=amax_ref.at[pl.ds(my, 1), :],
                    dst_ref=amax_ref.at[pl.ds(my, 1), :],
                    send_sem=ax_send_sems.at[k - 1],
                    recv_sem=ax_recv_sems.at[k - 1],
                    device_id=(dst,),
                    device_id_type=pl.DeviceIdType.MESH,
                )
                d.start()
                ax_sends.append(d)
            y = jnp.maximum(acc, 0.0)
        with jax.named_scope("amax_wait"):
            for k in range(1, N_DEV):
                src = lax.rem(my - k + N_DEV, N_DEV)
                recv = pltpu.make_async_remote_copy(
                    src_ref=amax_ref.at[pl.ds(src, 1), :],
                    dst_ref=amax_ref.at[pl.ds(src, 1), :],
                    send_sem=ax_send_sems.at[k - 1],
                    recv_sem=ax_recv_sems.at[k - 1],
                    device_id=(src,),
                    device_id_type=pl.DeviceIdType.MESH,
                )
                recv.wait_recv()

        with jax.named_scope("quant_out"):
            gmax = jnp.max(amax_ref[:, :])
            scale = gmax / 127.0
            q = jnp.clip(jnp.round(y * (127.0 / gmax)), -127.0, 127.0)
            out_ref[:, :] = q * scale

        with jax.named_scope("drain"):
            for d in sends:
                d.wait_send()
            for d in ax_sends:
                d.wait_send()

    return pl.pallas_call(
        body,
        out_shape=jax.ShapeDtypeStruct((m_glob // N_DEV, n), jnp.float32),
        in_specs=[
            pl.BlockSpec(memory_space=pltpu.MemorySpace.HBM),
            pl.BlockSpec(memory_space=pltpu.MemorySpace.HBM),
        ],
        out_specs=pl.BlockSpec(memory_space=pltpu.VMEM),
        scratch_shapes=[
            pltpu.VMEM((M_BLK, k_per), jnp.bfloat16),
            pltpu.VMEM((N_DEV - 1, M_BLK, K_BLK), jnp.int8),
            pltpu.VMEM((N_DEV, M_BLK, K_BLK), jnp.int8),
            pltpu.VMEM((N_DEV - 1, 1, K_BLK), jnp.float32),
            pltpu.VMEM((N_DEV, 1, K_BLK), jnp.float32),
            pltpu.VMEM((2, K_BLK, n), jnp.float32),
            pltpu.VMEM((2, K_BLK, n), jnp.bfloat16),
            pltpu.VMEM((4, M_BLK, k_per), jnp.float32),
            pltpu.VMEM((N_DEV, 128), jnp.float32),
            pltpu.SemaphoreType.DMA((N_DEV - 1,)),
            pltpu.SemaphoreType.DMA((N_DEV - 1,)),
            pltpu.SemaphoreType.DMA((N_DEV - 1,)),
            pltpu.SemaphoreType.DMA((N_DEV - 1,)),
            pltpu.SemaphoreType.DMA((2,)),
            pltpu.SemaphoreType.DMA((4,)),
            pltpu.SemaphoreType.DMA((N_DEV - 1,)),
            pltpu.SemaphoreType.DMA((N_DEV - 1,)),
        ],
        compiler_params=pltpu.CompilerParams(
            vmem_limit_bytes=64 * 1024 * 1024,
            collective_id=0,
        ),
    )(x, w_mat)


# device time: 39849 ns/iter; 1.1372x vs baseline; 1.1372x over previous
import jax
import jax.numpy as jnp
from jax import lax
from jax.experimental import pallas as pl
from jax.experimental.pallas import tpu as pltpu

N_DEV = 8
M_BLK = 512
K_BLK = 512

ORDER = [4, 1, 7, 3, 5, 2, 6]


def kernel(x, w_mat):
    m_glob, k_per = x.shape
    k_glob, n = w_mat.shape

    def body(x_ref, w_ref, out_ref,
             xb_ref, xq_send_ref, xq_ref, sc_send_ref, sc_ref,
             wv_ref, xs_ref, amax_ref,
             send_sems, recv_sems, sc_send_sems, sc_recv_sems,
             w_sems, xs_sems, ax_send_sems, ax_recv_sems):
        my = lax.axis_index("i")

        barrier = pltpu.get_barrier_semaphore()
        for k in range(1, N_DEV):
            pl.semaphore_signal(
                barrier, inc=1,
                device_id=(lax.rem(my + k, N_DEV),),
                device_id_type=pl.DeviceIdType.MESH,
            )

        dsts = [lax.rem(my + k, N_DEV) for k in ORDER]
        srcs = [my] + [lax.rem(my - k + N_DEV, N_DEV) for k in ORDER]

        def w_copy(j):
            return pltpu.make_async_copy(
                w_ref.at[pl.ds(srcs[j] * K_BLK, K_BLK), :],
                wv_ref.at[j % 2],
                w_sems.at[j % 2],
            )

        w_copy(0).start()
        w_copy(1).start()

        xblocks = dsts + [my]

        def x_copy(t):
            return pltpu.make_async_copy(
                x_ref.at[pl.ds(xblocks[t] * M_BLK, M_BLK), :],
                xs_ref.at[t % 2],
                xs_sems.at[t % 2],
            )

        x_copy(0).start()
        x_copy(1).start()

        pl.semaphore_wait(barrier, N_DEV - 1)

        sends = []
        for t in range(N_DEV):
            x_copy(t).wait()
            if t < N_DEV - 1:
                tile = xs_ref[t % 2]
                amax_c = jnp.maximum(
                    jnp.max(jnp.abs(tile), axis=0, keepdims=True), 1e-30)
                xq_send_ref[t] = jnp.clip(
                    jnp.round(tile * (127.0 / amax_c)), -127.0, 127.0
                ).astype(jnp.int8)
                sc_send_ref[t] = amax_c * (1.0 / 127.0)
            else:
                xb_ref[...] = xs_ref[t % 2].astype(jnp.bfloat16)
            if t + 2 < N_DEV:
                x_copy(t + 2).start()
            if t < N_DEV - 1:
                d = pltpu.make_async_remote_copy(
                    src_ref=xq_send_ref.at[t],
                    dst_ref=xq_ref.at[my],
                    send_sem=send_sems.at[t],
                    recv_sem=recv_sems.at[t],
                    device_id=(dsts[t],),
                    device_id_type=pl.DeviceIdType.MESH,
                )
                d.start()
                sends.append(d)
                s = pltpu.make_async_remote_copy(
                    src_ref=sc_send_ref.at[t],
                    dst_ref=sc_ref.at[my],
                    send_sem=sc_send_sems.at[t],
                    recv_sem=sc_recv_sems.at[t],
                    device_id=(dsts[t],),
                    device_id_type=pl.DeviceIdType.MESH,
                )
                s.start()
                sends.append(s)

        acc = jnp.zeros((M_BLK, n), jnp.float32)
        for j in range(N_DEV):
            if j == 0:
                x_tile = xb_ref[...]
            else:
                recv = pltpu.make_async_remote_copy(
                    src_ref=xq_send_ref.at[j - 1],
                    dst_ref=xq_ref.at[srcs[j]],
                    send_sem=send_sems.at[j - 1],
                    recv_sem=recv_sems.at[j - 1],
                    device_id=(srcs[j],),
                    device_id_type=pl.DeviceIdType.MESH,
                )
                recv.wait_recv()
                screcv = pltpu.make_async_remote_copy(
                    src_ref=sc_send_ref.at[j - 1],
                    dst_ref=sc_ref.at[srcs[j]],
                    send_sem=sc_send_sems.at[j - 1],
                    recv_sem=sc_recv_sems.at[j - 1],
                    device_id=(srcs[j],),
                    device_id_type=pl.DeviceIdType.MESH,
                )
                screcv.wait_recv()
                scale_row = sc_ref[srcs[j]]
                x_tile = (
                    xq_ref[srcs[j]].astype(jnp.float32) * scale_row
                ).astype(jnp.bfloat16)
            w_copy(j).wait()
            acc = acc + jnp.dot(
                x_tile,
                wv_ref[j % 2].astype(jnp.bfloat16),
                preferred_element_type=jnp.float32,
            )
            if j + 2 < N_DEV:
                w_copy(j + 2).start()

        y = jnp.maximum(acc, 0.0)

        amax_ref[pl.ds(my, 1), :] = jnp.full((1, 128), jnp.max(y), jnp.float32)
        ax_sends = []
        for k in range(1, N_DEV):
            dst = lax.rem(my + k, N_DEV)
            d = pltpu.make_async_remote_copy(
                src_ref=amax_ref.at[pl.ds(my, 1), :],
                dst_ref=amax_ref.at[pl.ds(my, 1), :],
                send_sem=ax_send_sems.at[k - 1],
                recv_sem=ax_recv_sems.at[k - 1],
                device_id=(dst,),
                device_id_type=pl.DeviceIdType.MESH,
            )
            d.start()
            ax_sends.append(d)
        for k in range(1, N_DEV):
            src = lax.rem(my - k + N_DEV, N_DEV)
            recv = pltpu.make_async_remote_copy(
                src_ref=amax_ref.at[pl.ds(src, 1), :],
                dst_ref=amax_ref.at[pl.ds(src, 1), :],
                send_sem=ax_send_sems.at[k - 1],
                recv_sem=ax_recv_sems.at[k - 1],
                device_id=(src,),
                device_id_type=pl.DeviceIdType.MESH,
            )
            recv.wait_recv()

        gmax = jnp.max(amax_ref[:, :])
        scale = gmax / 127.0
        q = jnp.clip(jnp.round(y * (127.0 / gmax)), -127.0, 127.0)
        out_ref[:, :] = q * scale

        for d in sends:
            d.wait_send()
        for d in ax_sends:
            d.wait_send()

    return pl.pallas_call(
        body,
        out_shape=jax.ShapeDtypeStruct((m_glob // N_DEV, n), jnp.float32),
        in_specs=[
            pl.BlockSpec(memory_space=pltpu.MemorySpace.HBM),
            pl.BlockSpec(memory_space=pltpu.MemorySpace.HBM),
        ],
        out_specs=pl.BlockSpec(memory_space=pltpu.VMEM),
        scratch_shapes=[
            pltpu.VMEM((M_BLK, k_per), jnp.bfloat16),
            pltpu.VMEM((N_DEV - 1, M_BLK, K_BLK), jnp.int8),
            pltpu.VMEM((N_DEV, M_BLK, K_BLK), jnp.int8),
            pltpu.VMEM((N_DEV - 1, 1, K_BLK), jnp.float32),
            pltpu.VMEM((N_DEV, 1, K_BLK), jnp.float32),
            pltpu.VMEM((2, K_BLK, n), jnp.float32),
            pltpu.VMEM((2, M_BLK, k_per), jnp.float32),
            pltpu.VMEM((N_DEV, 128), jnp.float32),
            pltpu.SemaphoreType.DMA((N_DEV - 1,)),
            pltpu.SemaphoreType.DMA((N_DEV - 1,)),
            pltpu.SemaphoreType.DMA((N_DEV - 1,)),
            pltpu.SemaphoreType.DMA((N_DEV - 1,)),
            pltpu.SemaphoreType.DMA((2,)),
            pltpu.SemaphoreType.DMA((2,)),
            pltpu.SemaphoreType.DMA((N_DEV - 1,)),
            pltpu.SemaphoreType.DMA((N_DEV - 1,)),
        ],
        compiler_params=pltpu.CompilerParams(
            vmem_limit_bytes=64 * 1024 * 1024,
            collective_id=0,
        ),
    )(x, w_mat)
